# baseline (speedup 1.0000x reference)
import functools
import math

import jax
import jax.numpy as jnp
from jax.experimental import pallas as pl
from jax.experimental.pallas import tpu as pltpu

_NN = (((1,), (0,)), ((), ()))
_NT = (((1,), (1,)), ((), ()))
_F32 = jnp.float32
_BF16 = jnp.bfloat16


def _merge_ln_proj_kernel(x_ref, g_ref, b_ref, w_ref, o_ref, *, eps, cin, c):
    blk = x_ref[...]
    tq, _, w_len, _ = blk.shape
    rows = tq * w_len
    toks = rows // 2
    c2 = 2 * c

    w0 = blk[:, 0].astype(_BF16).reshape(toks, c2)
    w1 = blk[:, 1].astype(_BF16).reshape(toks, c2)

    ones = jnp.ones((1, c2), _BF16)
    s = (jax.lax.dot_general(ones, w0, _NT, preferred_element_type=_F32)
         + jax.lax.dot_general(ones, w1, _NT, preferred_element_type=_F32))
    q = (jax.lax.dot_general(ones, w0 * w0, _NT, preferred_element_type=_F32)
         + jax.lax.dot_general(ones, w1 * w1, _NT, preferred_element_type=_F32))

    inv_cin = 1.0 / float(cin)
    mean = s * inv_cin
    var = q * inv_cin - mean * mean
    inv_row = jax.lax.rsqrt(var + eps)
    im_row = inv_row * mean
    inv = jnp.transpose(inv_row)
    im = jnp.transpose(im_row)

    g = g_ref[...]
    b = b_ref[...]
    w = w_ref[...]
    wb = w.astype(_BF16)
    wp = (w * g).astype(_BF16)

    ones_row = jnp.ones((1, cin), _BF16)
    gw = jax.lax.dot_general(ones_row, wp, _NT, preferred_element_type=_F32)
    bw = jax.lax.dot_general(b.astype(_BF16), wb, _NT, preferred_element_type=_F32)

    u = None
    for k, src in enumerate((w0[:, :c], w0[:, c:], w1[:, :c], w1[:, c:])):
        part = jax.lax.dot_general(src, wp[:, k * c:(k + 1) * c], _NT,
                                   preferred_element_type=_F32)
        u = part if u is None else u + part

    o_ref[...] = (u * inv - im * gw + bw).astype(o_ref.dtype)


def _run_one_core(x, gamma, beta, weight, H, W, eps):
    B, L, C = x.shape
    Ho, Wo = H // 2, W // 2
    Cin = 4 * C
    Cout = weight.shape[0]
    Nq = B * Ho
    N = Nq * Wo
    out_dtype = x.dtype

    xv = x.reshape(Nq, 2, W, C)
    g2 = gamma.reshape(1, Cin)
    b2 = beta.reshape(1, Cin)

    tq = 64
    grid = (pl.cdiv(Nq, tq),)

    cost = pl.CostEstimate(
        flops=int(2 * N * Cin * Cout),
        transcendentals=int(N),
        bytes_accessed=int(N * Cin * x.dtype.itemsize
                           + N * Cout * jnp.dtype(out_dtype).itemsize
                           + Cin * Cout * weight.dtype.itemsize),
    )

    out2d = pl.pallas_call(
        functools.partial(_merge_ln_proj_kernel, eps=eps, cin=Cin, c=C),
        out_shape=jax.ShapeDtypeStruct((N, Cout), out_dtype),
        grid=grid,
        in_specs=[
            pl.BlockSpec((tq, 2, W, C), lambda i: (i, 0, 0, 0)),
            pl.BlockSpec((1, Cin), lambda i: (0, 0)),
            pl.BlockSpec((1, Cin), lambda i: (0, 0)),
            pl.BlockSpec((Cout, Cin), lambda i: (0, 0)),
        ],
        out_specs=pl.BlockSpec((tq * Wo, Cout), lambda i: (i, 0)),
        compiler_params=pltpu.CompilerParams(
            dimension_semantics=("parallel",),
            vmem_limit_bytes=64 * 2**20,
        ),
        cost_estimate=cost,
    )(xv, g2, b2, weight)

    return out2d.reshape(B, Ho * Wo, Cout)


def kernel(x, gamma, beta, weight, *, eps=1e-5):
    B, L, C = x.shape
    H = W = math.isqrt(L)
    assert H * W == L and H % 2 == 0 and W % 2 == 0
    return _run_one_core(x, gamma, beta, weight, H, W, eps)


# speedup vs baseline: 1.0368x; 1.0368x over previous
import functools
import math

import jax
import jax.numpy as jnp
from jax.experimental import pallas as pl
from jax.experimental.pallas import tpu as pltpu

_NN = (((1,), (0,)), ((), ()))
_NT = (((1,), (1,)), ((), ()))
_F32 = jnp.float32
_BF16 = jnp.bfloat16


def _merge_ln_proj_kernel(x_ref, g_ref, b_ref, w_ref, o_ref, *, eps, cin, c):
    blk = x_ref[...]
    tq, _, w_len, _ = blk.shape
    rows = tq * w_len
    toks = rows // 2
    c2 = 2 * c

    w0 = blk[:, 0].astype(_BF16).reshape(toks, c2)
    w1 = blk[:, 1].astype(_BF16).reshape(toks, c2)

    ones = jnp.ones((1, c2), _BF16)
    s = (jax.lax.dot_general(ones, w0, _NT, preferred_element_type=_F32)
         + jax.lax.dot_general(ones, w1, _NT, preferred_element_type=_F32))
    q = (jax.lax.dot_general(ones, w0 * w0, _NT, preferred_element_type=_F32)
         + jax.lax.dot_general(ones, w1 * w1, _NT, preferred_element_type=_F32))

    inv_cin = 1.0 / float(cin)
    mean = s * inv_cin
    var = q * inv_cin - mean * mean
    inv_row = jax.lax.rsqrt(var + eps)
    im_row = inv_row * mean
    inv = jnp.transpose(inv_row)
    im = jnp.transpose(im_row)

    g = g_ref[...]
    b = b_ref[...]
    w = w_ref[...]
    wb = w.astype(_BF16)
    wp = (w * g).astype(_BF16)

    ones_row = jnp.ones((1, cin), _BF16)
    gw = jax.lax.dot_general(ones_row, wp, _NT, preferred_element_type=_F32)
    bw = jax.lax.dot_general(b.astype(_BF16), wb, _NT, preferred_element_type=_F32)

    u = None
    for k, src in enumerate((w0[:, :c], w0[:, c:], w1[:, :c], w1[:, c:])):
        part = jax.lax.dot_general(src, wp[:, k * c:(k + 1) * c], _NT,
                                   preferred_element_type=_F32)
        u = part if u is None else u + part

    o_ref[...] = (u * inv - im * gw + bw).astype(o_ref.dtype)


def _run_one_core(x, gamma, beta, weight, H, W, eps):
    B, L, C = x.shape
    Ho, Wo = H // 2, W // 2
    Cin = 4 * C
    Cout = weight.shape[0]
    Nq = B * Ho
    N = Nq * Wo
    out_dtype = x.dtype

    xv = x.reshape(Nq, 2, W, C)
    g2 = gamma.reshape(1, Cin)
    b2 = beta.reshape(1, Cin)

    tq = 128
    grid = (pl.cdiv(Nq, tq),)

    cost = pl.CostEstimate(
        flops=int(2 * N * Cin * Cout),
        transcendentals=int(N),
        bytes_accessed=int(N * Cin * x.dtype.itemsize
                           + N * Cout * jnp.dtype(out_dtype).itemsize
                           + Cin * Cout * weight.dtype.itemsize),
    )

    out2d = pl.pallas_call(
        functools.partial(_merge_ln_proj_kernel, eps=eps, cin=Cin, c=C),
        out_shape=jax.ShapeDtypeStruct((N, Cout), out_dtype),
        grid=grid,
        in_specs=[
            pl.BlockSpec((tq, 2, W, C), lambda i: (i, 0, 0, 0)),
            pl.BlockSpec((1, Cin), lambda i: (0, 0)),
            pl.BlockSpec((1, Cin), lambda i: (0, 0)),
            pl.BlockSpec((Cout, Cin), lambda i: (0, 0)),
        ],
        out_specs=pl.BlockSpec((tq * Wo, Cout), lambda i: (i, 0)),
        compiler_params=pltpu.CompilerParams(
            dimension_semantics=("parallel",),
            vmem_limit_bytes=64 * 2**20,
        ),
        cost_estimate=cost,
    )(xv, g2, b2, weight)

    return out2d.reshape(B, Ho * Wo, Cout)


def kernel(x, gamma, beta, weight, *, eps=1e-5):
    B, L, C = x.shape
    H = W = math.isqrt(L)
    assert H * W == L and H % 2 == 0 and W % 2 == 0
    return _run_one_core(x, gamma, beta, weight, H, W, eps)
